# baseline (speedup 1.0000x reference)
import functools

import jax
import jax.numpy as jnp
from jax import lax
from jax.experimental import pallas as pl
from jax.experimental.pallas import tpu as pltpu
from jax.experimental.pallas import tpu_sc as plsc

N = 100000
S = 64
L = 16384
EPS = 1e-12
LN2 = 0.6931471805599453
D = 80
NW = 32
CHUNK = L // NW
SUB = 128
LB = 2048


def _log1mexp(x):
    x = jnp.maximum(x, 1e-10)
    em = jnp.where(x < 0.01,
                   x * (1.0 - x * (0.5 - x * (1.0 / 6.0))),
                   1.0 - jnp.exp(-x))
    return jnp.where(
        x > LN2,
        jnp.log1p(-jnp.exp(-x)),
        jnp.log(em + EPS),
    )


def _sc_gather_body(table_hbm, i1_hbm, i2_hbm, o1_hbm, o2_hbm,
                    idx1_v, idx2_v, rows1_v, rows2_v, sem):
    wid = lax.axis_index("s") * 2 + lax.axis_index("c")
    base = wid * CHUNK
    pltpu.sync_copy(i1_hbm.at[pl.ds(base, CHUNK)], idx1_v)
    pltpu.sync_copy(i2_hbm.at[pl.ds(base, CHUNK)], idx2_v)
    copies = []
    for j in range(CHUNK // SUB):
        sl = pl.ds(j * SUB, SUB)
        copies.append(pltpu.async_copy(table_hbm.at[idx1_v.at[sl]],
                                       rows1_v.at[sl], sem))
        copies.append(pltpu.async_copy(table_hbm.at[idx2_v.at[sl]],
                                       rows2_v.at[sl], sem))
    for cp in copies:
        cp.wait()
    pltpu.sync_copy(rows1_v, o1_hbm.at[pl.ds(base, CHUNK)])
    pltpu.sync_copy(rows2_v, o2_hbm.at[pl.ds(base, CHUNK)])


def _sc_gather(table, idx1, idx2):
    mesh = plsc.VectorSubcoreMesh(core_axis_name="c", subcore_axis_name="s")
    f = functools.partial(
        pl.kernel, mesh=mesh,
        out_type=(jax.ShapeDtypeStruct((L, D), jnp.float32),
                  jax.ShapeDtypeStruct((L, D), jnp.float32)),
        scratch_types=[
            pltpu.VMEM((CHUNK,), jnp.int32),
            pltpu.VMEM((CHUNK,), jnp.int32),
            pltpu.VMEM((CHUNK, D), jnp.float32),
            pltpu.VMEM((CHUNK, D), jnp.float32),
            pltpu.SemaphoreType.DMA,
        ],
        compiler_params=pltpu.CompilerParams(use_tc_tiling_on_sc=False),
    )(_sc_gather_body)
    return f(table, idx1, idx2)


def _tc_body(scal_ref, epsR_ref, epsA_ref, epsT_ref,
             g1_ref, g2_ref, w_ref, out_ref):
    Rc = scal_ref[0]
    Rsc = scal_ref[1]
    ac = scal_ref[2]
    asc = scal_ref[3]
    T0 = scal_ref[4]
    T1 = scal_ref[5]

    epsR = epsR_ref[...]
    epsA = epsA_ref[...]
    epsT = epsT_ref[...]

    R_s = jnp.exp(Rc) * jnp.exp(Rsc) * jnp.exp(0.1 * epsR)
    alpha_s = jnp.exp(ac) * jnp.exp(asc) * jnp.exp(0.1 * epsA)
    T_x0 = jnp.exp(T0)
    T_x1 = jnp.exp(T1)
    z = epsT + jnp.log(T_x0 / (T_x1 + EPS))
    T_samp = 1.0 / (1.0 + jnp.exp(-z))
    alpha_R = _log1mexp(alpha_s * R_s)
    lt = jnp.log1p(T_samp)

    g1 = g1_ref[...]
    g2 = g2_ref[...]
    eps1 = g1[:, 0:S]
    a1 = g1[:, S:S + 1]
    b1 = g1[:, S + 1:S + 2]
    eps2 = g2[:, 0:S]
    a2 = g2[:, S:S + 1]
    b2 = g2[:, S + 1:S + 2]

    s1 = jnp.exp(b1)
    s2 = jnp.exp(b2)
    hi = R_s - EPS
    r_i = jnp.minimum(jnp.maximum(a1 + s1 * eps1, EPS), hi)
    r_j = jnp.minimum(jnp.maximum(a2 + s2 * eps2, EPS), hi)

    alpha_r_i = _log1mexp(2.0 * alpha_s * r_i)
    a_R_ri = -alpha_s * (R_s - r_i)
    a_R_rj = -alpha_s * (R_s - r_j)

    s1e = s1 + EPS
    q = (-0.5 * ((r_i - a1) / s1e) ** 2 - jnp.log(s1e)
         - 0.5 * jnp.log(2.0 * jnp.pi))

    edges = jnp.where(w_ref[...] > 0, 1.0, 0.0)
    out_ref[...] = (edges * (alpha_r_i + a_R_ri + a_R_rj)
                    - alpha_R + q - lt)


def _tc_compute(scal, epsR, epsA, epsT, g1, g2, w2d):
    grid = (L // LB,)
    return pl.pallas_call(
        _tc_body,
        grid=grid,
        in_specs=[
            pl.BlockSpec(memory_space=pltpu.SMEM),
            pl.BlockSpec((1, S), lambda i: (0, 0)),
            pl.BlockSpec((1, S), lambda i: (0, 0)),
            pl.BlockSpec((1, S), lambda i: (0, 0)),
            pl.BlockSpec((LB, D), lambda i: (i, 0)),
            pl.BlockSpec((LB, D), lambda i: (i, 0)),
            pl.BlockSpec((LB, 1), lambda i: (i, 0)),
        ],
        out_specs=pl.BlockSpec((LB, S), lambda i: (i, 0)),
        out_shape=jax.ShapeDtypeStruct((L, S), jnp.float32),
    )(scal, epsR, epsA, epsT, g1, g2, w2d)


def kernel(idx1, idx2, weights, rs_loc, rs_scale, phis_loc, phis_scale,
           R_conc, R_scale, T, alpha_conc, alpha_scale,
           eps_R, eps_T, eps_alpha, eps_r):
    idx1 = idx1.astype(jnp.int32)
    idx2 = idx2.astype(jnp.int32)
    table = jnp.concatenate(
        [eps_r.T, rs_loc[:, None], rs_scale[:, None],
         jnp.zeros((N, D - S - 2), jnp.float32)], axis=1)
    g1, g2 = _sc_gather(table, idx1, idx2)
    scal = jnp.stack([R_conc, R_scale, alpha_conc, alpha_scale,
                      T[0], T[1]]).astype(jnp.float32)
    out_t = _tc_compute(scal, eps_R[None, :], eps_alpha[None, :],
                        eps_T[None, :], g1, g2, weights[:, None])
    return out_t.T


# speedup vs baseline: 1.3084x; 1.3084x over previous
import functools

import jax
import jax.numpy as jnp
from jax import lax
from jax.experimental import pallas as pl
from jax.experimental.pallas import tpu as pltpu
from jax.experimental.pallas import tpu_sc as plsc

N = 100000
S = 64
L = 16384
EPS = 1e-12
LN2 = 0.6931471805599453
D = 128
NW = 32
CHUNK = L // NW
SUB = 128
LB = 1024


def _log1mexp(x):
    x = jnp.maximum(x, 1e-10)
    em = jnp.where(x < 0.01,
                   x * (1.0 - x * (0.5 - x * (1.0 / 6.0))),
                   1.0 - jnp.exp(-x))
    return jnp.where(
        x > LN2,
        jnp.log1p(-jnp.exp(-x)),
        jnp.log(em + EPS),
    )


def _sc_gather_body(table_hbm, i1_hbm, i2_hbm, o1_hbm, o2_hbm,
                    idx1_v, idx2_v, rows_v, sem):
    wid = lax.axis_index("s") * 2 + lax.axis_index("c")
    base = wid * CHUNK
    pltpu.sync_copy(i1_hbm.at[pl.ds(base, CHUNK)], idx1_v)
    pltpu.sync_copy(i2_hbm.at[pl.ds(base, CHUNK)], idx2_v)
    copies = []
    for j in range(CHUNK // SUB):
        sl = pl.ds(j * SUB, SUB)
        copies.append(pltpu.async_copy(table_hbm.at[idx1_v.at[sl]],
                                       rows_v.at[sl], sem))
    for cp in copies:
        cp.wait()
    pltpu.sync_copy(rows_v, o1_hbm.at[pl.ds(base, CHUNK)])
    copies = []
    for j in range(CHUNK // SUB):
        sl = pl.ds(j * SUB, SUB)
        copies.append(pltpu.async_copy(table_hbm.at[idx2_v.at[sl]],
                                       rows_v.at[sl], sem))
    for cp in copies:
        cp.wait()
    pltpu.sync_copy(rows_v, o2_hbm.at[pl.ds(base, CHUNK)])


def _sc_gather(table, idx1, idx2):
    mesh = plsc.VectorSubcoreMesh(core_axis_name="c", subcore_axis_name="s")
    f = functools.partial(
        pl.kernel, mesh=mesh,
        out_type=(jax.ShapeDtypeStruct((L, D), jnp.float32),
                  jax.ShapeDtypeStruct((L, D), jnp.float32)),
        scratch_types=[
            pltpu.VMEM((CHUNK,), jnp.int32),
            pltpu.VMEM((CHUNK,), jnp.int32),
            pltpu.VMEM((CHUNK, D), jnp.float32),
            pltpu.SemaphoreType.DMA,
        ],
    )(_sc_gather_body)
    return f(table, idx1, idx2)


def _tc_body(scal_ref, epsR_ref, epsA_ref, epsT_ref,
             g1_ref, g2_ref, w_ref, out_ref):
    Rc = scal_ref[0]
    Rsc = scal_ref[1]
    ac = scal_ref[2]
    asc = scal_ref[3]
    T0 = scal_ref[4]
    T1 = scal_ref[5]

    epsR = epsR_ref[...]
    epsA = epsA_ref[...]
    epsT = epsT_ref[...]

    R_s = jnp.exp(Rc) * jnp.exp(Rsc) * jnp.exp(0.1 * epsR)
    alpha_s = jnp.exp(ac) * jnp.exp(asc) * jnp.exp(0.1 * epsA)
    T_x0 = jnp.exp(T0)
    T_x1 = jnp.exp(T1)
    z = epsT + jnp.log(T_x0 / (T_x1 + EPS))
    T_samp = 1.0 / (1.0 + jnp.exp(-z))
    alpha_R = _log1mexp(alpha_s * R_s)
    lt = jnp.log1p(T_samp)

    eps1 = g1_ref[:, 0:S]
    a1 = g1_ref[:, S:S + 1]
    b1 = g1_ref[:, S + 1:S + 2]
    eps2 = g2_ref[:, 0:S]
    a2 = g2_ref[:, S:S + 1]
    b2 = g2_ref[:, S + 1:S + 2]

    s1 = jnp.exp(b1)
    s2 = jnp.exp(b2)
    hi = R_s - EPS
    r_i = jnp.minimum(jnp.maximum(a1 + s1 * eps1, EPS), hi)
    r_j = jnp.minimum(jnp.maximum(a2 + s2 * eps2, EPS), hi)

    alpha_r_i = _log1mexp(2.0 * alpha_s * r_i)
    a_R_ri = -alpha_s * (R_s - r_i)
    a_R_rj = -alpha_s * (R_s - r_j)

    s1e = s1 + EPS
    inv_s1 = 1.0 / s1e
    log_s1 = jnp.log(s1e)
    q = (-0.5 * ((r_i - a1) * inv_s1) ** 2 - log_s1
         - 0.5 * jnp.log(2.0 * jnp.pi))

    edges = jnp.where(w_ref[...] > 0, 1.0, 0.0)
    out_ref[...] = (edges * (alpha_r_i + a_R_ri + a_R_rj)
                    - alpha_R + q - lt)


def _tc_compute(scal, epsR, epsA, epsT, g1, g2, w2d):
    grid = (L // LB,)
    return pl.pallas_call(
        _tc_body,
        grid=grid,
        in_specs=[
            pl.BlockSpec(memory_space=pltpu.SMEM),
            pl.BlockSpec((1, S), lambda i: (0, 0)),
            pl.BlockSpec((1, S), lambda i: (0, 0)),
            pl.BlockSpec((1, S), lambda i: (0, 0)),
            pl.BlockSpec((LB, D), lambda i: (i, 0)),
            pl.BlockSpec((LB, D), lambda i: (i, 0)),
            pl.BlockSpec((LB, 1), lambda i: (i, 0)),
        ],
        out_specs=pl.BlockSpec((LB, S), lambda i: (i, 0)),
        out_shape=jax.ShapeDtypeStruct((L, S), jnp.float32),
    )(scal, epsR, epsA, epsT, g1, g2, w2d)


def kernel(idx1, idx2, weights, rs_loc, rs_scale, phis_loc, phis_scale,
           R_conc, R_scale, T, alpha_conc, alpha_scale,
           eps_R, eps_T, eps_alpha, eps_r):
    idx1 = idx1.astype(jnp.int32)
    idx2 = idx2.astype(jnp.int32)
    table = jnp.concatenate(
        [eps_r.T, rs_loc[:, None], rs_scale[:, None],
         jnp.zeros((N, D - S - 2), jnp.float32)], axis=1)
    g1, g2 = _sc_gather(table, idx1, idx2)
    scal = jnp.stack([R_conc, R_scale, alpha_conc, alpha_scale,
                      T[0], T[1]]).astype(jnp.float32)
    out_t = _tc_compute(scal, eps_R[None, :], eps_alpha[None, :],
                        eps_T[None, :], g1, g2, weights[:, None])
    return out_t.T


# speedup vs baseline: 1.8617x; 1.4229x over previous
import functools

import jax
import jax.numpy as jnp
from jax import lax
from jax.experimental import pallas as pl
from jax.experimental.pallas import tpu as pltpu
from jax.experimental.pallas import tpu_sc as plsc

N = 100000
S = 64
L = 16384
EPS = 1e-12
LN2 = 0.6931471805599453
D = 128
NW = 32
CHUNK = L // NW
SUB = 128
LB = 1024


def _log1mexp(x):
    x = jnp.maximum(x, 1e-10)
    em = jnp.where(x < 0.01,
                   x * (1.0 - x * (0.5 - x * (1.0 / 6.0))),
                   1.0 - jnp.exp(-x))
    return jnp.log(em + EPS)


NB = 1024


def _tbuild_body(eps_ref, rsl_ref, rss_ref, eye_ref, out_ref):
    x = jnp.concatenate([eps_ref[...],
                         rsl_ref[...].reshape(1, NB),
                         rss_ref[...].reshape(1, NB)], axis=0)
    out_ref[...] = lax.dot_general(
        x, eye_ref[...],
        dimension_numbers=(((0,), (0,)), ((), ())),
        preferred_element_type=jnp.float32,
        precision=lax.Precision.HIGHEST)


def _build_table(eps_r, rs_loc, rs_scale):
    eye = jnp.eye(S + 2, D, dtype=jnp.float32)
    return pl.pallas_call(
        _tbuild_body,
        grid=(pl.cdiv(N, NB),),
        in_specs=[
            pl.BlockSpec((S, NB), lambda i: (0, i)),
            pl.BlockSpec((NB,), lambda i: (i,)),
            pl.BlockSpec((NB,), lambda i: (i,)),
            pl.BlockSpec((S + 2, D), lambda i: (0, 0)),
        ],
        out_specs=pl.BlockSpec((NB, D), lambda i: (i, 0)),
        out_shape=jax.ShapeDtypeStruct((N, D), jnp.float32),
    )(eps_r, rs_loc, rs_scale, eye)


def _sc_gather_body(table_hbm, i1_hbm, i2_hbm, o1_hbm, o2_hbm,
                    idx1_v, idx2_v, rows_v, sem):
    wid = lax.axis_index("s") * 2 + lax.axis_index("c")
    base = wid * CHUNK
    pltpu.sync_copy(i1_hbm.at[pl.ds(base, CHUNK)], idx1_v)
    pltpu.sync_copy(i2_hbm.at[pl.ds(base, CHUNK)], idx2_v)
    copies = []
    for j in range(CHUNK // SUB):
        sl = pl.ds(j * SUB, SUB)
        copies.append(pltpu.async_copy(table_hbm.at[idx1_v.at[sl]],
                                       rows_v.at[sl], sem))
    for cp in copies:
        cp.wait()
    pltpu.sync_copy(rows_v, o1_hbm.at[pl.ds(base, CHUNK)])
    copies = []
    for j in range(CHUNK // SUB):
        sl = pl.ds(j * SUB, SUB)
        copies.append(pltpu.async_copy(table_hbm.at[idx2_v.at[sl]],
                                       rows_v.at[sl], sem))
    for cp in copies:
        cp.wait()
    pltpu.sync_copy(rows_v, o2_hbm.at[pl.ds(base, CHUNK)])


def _sc_gather(table, idx1, idx2):
    mesh = plsc.VectorSubcoreMesh(core_axis_name="c", subcore_axis_name="s")
    f = functools.partial(
        pl.kernel, mesh=mesh,
        out_type=(jax.ShapeDtypeStruct((L, D), jnp.float32),
                  jax.ShapeDtypeStruct((L, D), jnp.float32)),
        scratch_types=[
            pltpu.VMEM((CHUNK,), jnp.int32),
            pltpu.VMEM((CHUNK,), jnp.int32),
            pltpu.VMEM((CHUNK, D), jnp.float32),
            pltpu.SemaphoreType.DMA,
        ],
    )(_sc_gather_body)
    return f(table, idx1, idx2)


def _tc_body(scal_ref, epsR_ref, epsA_ref, epsT_ref,
             g1_ref, g2_ref, w_ref, out_ref):
    Rc = scal_ref[0]
    Rsc = scal_ref[1]
    ac = scal_ref[2]
    asc = scal_ref[3]
    T0 = scal_ref[4]
    T1 = scal_ref[5]

    epsR = epsR_ref[...]
    epsA = epsA_ref[...]
    epsT = epsT_ref[...]

    R_s = jnp.exp(Rc) * jnp.exp(Rsc) * jnp.exp(0.1 * epsR)
    alpha_s = jnp.exp(ac) * jnp.exp(asc) * jnp.exp(0.1 * epsA)
    T_x0 = jnp.exp(T0)
    T_x1 = jnp.exp(T1)
    z = epsT + jnp.log(T_x0 / (T_x1 + EPS))
    T_samp = 1.0 / (1.0 + jnp.exp(-z))
    alpha_R = _log1mexp(alpha_s * R_s)
    lt = jnp.log1p(T_samp)

    eps1 = g1_ref[:, 0:S]
    a1 = g1_ref[:, S:S + 1]
    b1 = g1_ref[:, S + 1:S + 2]
    eps2 = g2_ref[:, 0:S]
    a2 = g2_ref[:, S:S + 1]
    b2 = g2_ref[:, S + 1:S + 2]

    s1 = jnp.exp(b1)
    s2 = jnp.exp(b2)
    hi = R_s - EPS
    r_i = jnp.minimum(jnp.maximum(a1 + s1 * eps1, EPS), hi)
    r_j = jnp.minimum(jnp.maximum(a2 + s2 * eps2, EPS), hi)

    alpha_r_i = _log1mexp(2.0 * alpha_s * r_i)
    a_R_ri = -alpha_s * (R_s - r_i)
    a_R_rj = -alpha_s * (R_s - r_j)

    s1e = s1 + EPS
    inv_s1 = 1.0 / s1e
    log_s1 = jnp.log(s1e)
    q = (-0.5 * ((r_i - a1) * inv_s1) ** 2 - log_s1
         - 0.5 * jnp.log(2.0 * jnp.pi))

    edges = jnp.where(w_ref[...] > 0, 1.0, 0.0)
    out_ref[...] = (edges * (alpha_r_i + a_R_ri + a_R_rj)
                    - alpha_R + q - lt)


def _tc_compute(scal, epsR, epsA, epsT, g1, g2, w2d):
    grid = (L // LB,)
    return pl.pallas_call(
        _tc_body,
        grid=grid,
        in_specs=[
            pl.BlockSpec(memory_space=pltpu.SMEM),
            pl.BlockSpec((1, S), lambda i: (0, 0)),
            pl.BlockSpec((1, S), lambda i: (0, 0)),
            pl.BlockSpec((1, S), lambda i: (0, 0)),
            pl.BlockSpec((LB, D), lambda i: (i, 0)),
            pl.BlockSpec((LB, D), lambda i: (i, 0)),
            pl.BlockSpec((LB, 1), lambda i: (i, 0)),
        ],
        out_specs=pl.BlockSpec((LB, S), lambda i: (i, 0)),
        out_shape=jax.ShapeDtypeStruct((L, S), jnp.float32),
    )(scal, epsR, epsA, epsT, g1, g2, w2d)


def kernel(idx1, idx2, weights, rs_loc, rs_scale, phis_loc, phis_scale,
           R_conc, R_scale, T, alpha_conc, alpha_scale,
           eps_R, eps_T, eps_alpha, eps_r):
    idx1 = idx1.astype(jnp.int32)
    idx2 = idx2.astype(jnp.int32)
    table = _build_table(eps_r, rs_loc, rs_scale)
    g1, g2 = _sc_gather(table, idx1, idx2)
    scal = jnp.stack([R_conc, R_scale, alpha_conc, alpha_scale,
                      T[0], T[1]]).astype(jnp.float32)
    out_t = _tc_compute(scal, eps_R[None, :], eps_alpha[None, :],
                        eps_T[None, :], g1, g2, weights[:, None])
    return out_t.T


# speedup vs baseline: 2.2299x; 1.1978x over previous
import functools

import jax
import jax.numpy as jnp
from jax import lax
from jax.experimental import pallas as pl
from jax.experimental.pallas import tpu as pltpu
from jax.experimental.pallas import tpu_sc as plsc

N = 100000
S = 64
L = 16384
EPS = 1e-12
LN2 = 0.6931471805599453
D = 128
NW = 32
CHUNK = L // NW
SUB = 128
LB = 1024


def _log1mexp(x):
    x = jnp.maximum(x, 1e-10)
    em = jnp.where(x < 0.01,
                   x * (1.0 - x * (0.5 - x * (1.0 / 6.0))),
                   1.0 - jnp.exp(-x))
    return jnp.log(em + EPS)


NB = 2048
DW = 72


def _tbuild_body(eps_ref, rsl_ref, rss_ref, eye_ref, out_ref):
    x = jnp.concatenate([eps_ref[...],
                         rsl_ref[...].reshape(1, NB),
                         rss_ref[...].reshape(1, NB)], axis=0)
    out_ref[:, 0:DW] = lax.dot_general(
        x, eye_ref[...],
        dimension_numbers=(((0,), (0,)), ((), ())),
        preferred_element_type=jnp.float32,
        precision=lax.Precision.HIGHEST)


def _build_table(eps_r, rs_loc, rs_scale):
    eye = jnp.eye(S + 2, DW, dtype=jnp.float32)
    return pl.pallas_call(
        _tbuild_body,
        grid=(pl.cdiv(N, NB),),
        in_specs=[
            pl.BlockSpec((S, NB), lambda i: (0, i)),
            pl.BlockSpec((NB,), lambda i: (i,)),
            pl.BlockSpec((NB,), lambda i: (i,)),
            pl.BlockSpec((S + 2, DW), lambda i: (0, 0)),
        ],
        out_specs=pl.BlockSpec((NB, D), lambda i: (i, 0)),
        out_shape=jax.ShapeDtypeStruct((N, D), jnp.float32),
    )(eps_r, rs_loc, rs_scale, eye)


def _sc_gather_body(table_hbm, i1_hbm, i2_hbm, o1_hbm, o2_hbm,
                    idx1_v, idx2_v, rows_v, sem):
    wid = lax.axis_index("s") * 2 + lax.axis_index("c")
    base = wid * CHUNK
    pltpu.sync_copy(i1_hbm.at[pl.ds(base, CHUNK)], idx1_v)
    pltpu.sync_copy(i2_hbm.at[pl.ds(base, CHUNK)], idx2_v)
    copies = []
    for j in range(CHUNK // SUB):
        sl = pl.ds(j * SUB, SUB)
        copies.append(pltpu.async_copy(table_hbm.at[idx1_v.at[sl]],
                                       rows_v.at[sl], sem))
    for cp in copies:
        cp.wait()
    pltpu.sync_copy(rows_v, o1_hbm.at[pl.ds(base, CHUNK)])
    copies = []
    for j in range(CHUNK // SUB):
        sl = pl.ds(j * SUB, SUB)
        copies.append(pltpu.async_copy(table_hbm.at[idx2_v.at[sl]],
                                       rows_v.at[sl], sem))
    for cp in copies:
        cp.wait()
    pltpu.sync_copy(rows_v, o2_hbm.at[pl.ds(base, CHUNK)])


def _sc_gather(table, idx1, idx2):
    mesh = plsc.VectorSubcoreMesh(core_axis_name="c", subcore_axis_name="s")
    f = functools.partial(
        pl.kernel, mesh=mesh,
        out_type=(jax.ShapeDtypeStruct((L, D), jnp.float32),
                  jax.ShapeDtypeStruct((L, D), jnp.float32)),
        scratch_types=[
            pltpu.VMEM((CHUNK,), jnp.int32),
            pltpu.VMEM((CHUNK,), jnp.int32),
            pltpu.VMEM((CHUNK, D), jnp.float32),
            pltpu.SemaphoreType.DMA,
        ],
    )(_sc_gather_body)
    return f(table, idx1, idx2)


def _tc_body(Rc_ref, Rsc_ref, ac_ref, asc_ref, T_ref, epsR_ref, epsA_ref,
             epsT_ref, g1_ref, g2_ref, w_ref, out_ref):
    Rc = Rc_ref[0]
    Rsc = Rsc_ref[0]
    ac = ac_ref[0]
    asc = asc_ref[0]
    T0 = T_ref[0]
    T1 = T_ref[1]

    epsR = epsR_ref[...]
    epsA = epsA_ref[...]
    epsT = epsT_ref[...]

    R_s = jnp.exp(Rc) * jnp.exp(Rsc) * jnp.exp(0.1 * epsR)
    alpha_s = jnp.exp(ac) * jnp.exp(asc) * jnp.exp(0.1 * epsA)
    T_x0 = jnp.exp(T0)
    T_x1 = jnp.exp(T1)
    z = epsT + jnp.log(T_x0 / (T_x1 + EPS))
    T_samp = 1.0 / (1.0 + jnp.exp(-z))
    alpha_R = _log1mexp(alpha_s * R_s)
    lt = jnp.log1p(T_samp)

    eps1 = g1_ref[:, 0:S]
    a1 = g1_ref[:, S:S + 1]
    b1 = g1_ref[:, S + 1:S + 2]
    eps2 = g2_ref[:, 0:S]
    a2 = g2_ref[:, S:S + 1]
    b2 = g2_ref[:, S + 1:S + 2]

    s1 = jnp.exp(b1)
    s2 = jnp.exp(b2)
    hi = R_s - EPS
    r_i = jnp.minimum(jnp.maximum(a1 + s1 * eps1, EPS), hi)
    r_j = jnp.minimum(jnp.maximum(a2 + s2 * eps2, EPS), hi)

    alpha_r_i = _log1mexp(2.0 * alpha_s * r_i)
    a_R_ri = -alpha_s * (R_s - r_i)
    a_R_rj = -alpha_s * (R_s - r_j)

    s1e = s1 + EPS
    inv_s1 = 1.0 / s1e
    log_s1 = jnp.log(s1e)
    q = (-0.5 * ((r_i - a1) * inv_s1) ** 2 - log_s1
         - 0.5 * jnp.log(2.0 * jnp.pi))

    edges = jnp.where(w_ref[...] > 0, 1.0, 0.0)
    out_ref[...] = (edges * (alpha_r_i + a_R_ri + a_R_rj)
                    - alpha_R + q - lt)


def _tc_compute(Rc, Rsc, ac, asc, T, epsR, epsA, epsT, g1, g2, w2d):
    grid = (L // LB,)
    return pl.pallas_call(
        _tc_body,
        grid=grid,
        in_specs=[
            pl.BlockSpec(memory_space=pltpu.SMEM),
            pl.BlockSpec(memory_space=pltpu.SMEM),
            pl.BlockSpec(memory_space=pltpu.SMEM),
            pl.BlockSpec(memory_space=pltpu.SMEM),
            pl.BlockSpec(memory_space=pltpu.SMEM),
            pl.BlockSpec((1, S), lambda i: (0, 0)),
            pl.BlockSpec((1, S), lambda i: (0, 0)),
            pl.BlockSpec((1, S), lambda i: (0, 0)),
            pl.BlockSpec((LB, D), lambda i: (i, 0)),
            pl.BlockSpec((LB, D), lambda i: (i, 0)),
            pl.BlockSpec((LB, 1), lambda i: (i, 0)),
        ],
        out_specs=pl.BlockSpec((LB, S), lambda i: (i, 0)),
        out_shape=jax.ShapeDtypeStruct((L, S), jnp.float32),
    )(Rc, Rsc, ac, asc, T, epsR, epsA, epsT, g1, g2, w2d)


def kernel(idx1, idx2, weights, rs_loc, rs_scale, phis_loc, phis_scale,
           R_conc, R_scale, T, alpha_conc, alpha_scale,
           eps_R, eps_T, eps_alpha, eps_r):
    idx1 = idx1.astype(jnp.int32)
    idx2 = idx2.astype(jnp.int32)
    table = _build_table(eps_r, rs_loc, rs_scale)
    g1, g2 = _sc_gather(table, idx1, idx2)
    out_t = _tc_compute(R_conc.reshape(1), R_scale.reshape(1),
                        alpha_conc.reshape(1), alpha_scale.reshape(1), T,
                        eps_R[None, :], eps_alpha[None, :],
                        eps_T[None, :], g1, g2, weights[:, None])
    return out_t.T


# speedup vs baseline: 2.5964x; 1.1644x over previous
import functools

import jax
import jax.numpy as jnp
from jax import lax
from jax.experimental import pallas as pl
from jax.experimental.pallas import tpu as pltpu
from jax.experimental.pallas import tpu_sc as plsc

N = 100000
S = 64
L = 16384
EPS = 1e-12
LN2 = 0.6931471805599453
D = 128
NW = 32
CHUNK = L // NW
SUB = 128
LB = 2048


def _log1mexp(x):
    x = jnp.maximum(x, 1e-10)
    em = jnp.where(x < 0.01,
                   x * (1.0 - x * (0.5 - x * (1.0 / 6.0))),
                   1.0 - jnp.exp(-x))
    return jnp.log(em + EPS)


NB = 4096
DW = 72


def _tbuild_body(eps_ref, rsl_ref, rss_ref, eye_ref, out_ref):
    x = jnp.concatenate([eps_ref[...],
                         rsl_ref[...].reshape(1, NB),
                         rss_ref[...].reshape(1, NB)], axis=0)
    out_ref[:, 0:DW] = lax.dot_general(
        x, eye_ref[...],
        dimension_numbers=(((0,), (0,)), ((), ())),
        preferred_element_type=jnp.float32,
        precision=lax.Precision.HIGHEST)


def _build_table(eps_r, rs_loc, rs_scale):
    eye = jnp.eye(S + 2, DW, dtype=jnp.float32)
    return pl.pallas_call(
        _tbuild_body,
        grid=(pl.cdiv(N, NB),),
        in_specs=[
            pl.BlockSpec((S, NB), lambda i: (0, i)),
            pl.BlockSpec((NB,), lambda i: (i,)),
            pl.BlockSpec((NB,), lambda i: (i,)),
            pl.BlockSpec((S + 2, DW), lambda i: (0, 0)),
        ],
        out_specs=pl.BlockSpec((NB, D), lambda i: (i, 0)),
        out_shape=jax.ShapeDtypeStruct((N, D), jnp.float32),
    )(eps_r, rs_loc, rs_scale, eye)


def _sc_gather_body(table_hbm, i1_hbm, i2_hbm, o1_hbm, o2_hbm,
                    idx1_v, idx2_v, rows_v, sem):
    wid = lax.axis_index("s") * 2 + lax.axis_index("c")
    base = wid * CHUNK
    pltpu.sync_copy(i1_hbm.at[pl.ds(base, CHUNK)], idx1_v)
    pltpu.sync_copy(i2_hbm.at[pl.ds(base, CHUNK)], idx2_v)
    copies = []
    for j in range(CHUNK // SUB):
        sl = pl.ds(j * SUB, SUB)
        copies.append(pltpu.async_copy(table_hbm.at[idx1_v.at[sl]],
                                       rows_v.at[sl], sem))
    for cp in copies:
        cp.wait()
    pltpu.sync_copy(rows_v, o1_hbm.at[pl.ds(base, CHUNK)])
    copies = []
    for j in range(CHUNK // SUB):
        sl = pl.ds(j * SUB, SUB)
        copies.append(pltpu.async_copy(table_hbm.at[idx2_v.at[sl]],
                                       rows_v.at[sl], sem))
    for cp in copies:
        cp.wait()
    pltpu.sync_copy(rows_v, o2_hbm.at[pl.ds(base, CHUNK)])


def _sc_gather(table, idx1, idx2):
    mesh = plsc.VectorSubcoreMesh(core_axis_name="c", subcore_axis_name="s")
    f = functools.partial(
        pl.kernel, mesh=mesh,
        out_type=(jax.ShapeDtypeStruct((L, D), jnp.float32),
                  jax.ShapeDtypeStruct((L, D), jnp.float32)),
        scratch_types=[
            pltpu.VMEM((CHUNK,), jnp.int32),
            pltpu.VMEM((CHUNK,), jnp.int32),
            pltpu.VMEM((CHUNK, D), jnp.float32),
            pltpu.SemaphoreType.DMA,
        ],
    )(_sc_gather_body)
    return f(table, idx1, idx2)


def _tc_body(Rc_ref, Rsc_ref, ac_ref, asc_ref, T_ref, epsR_ref, epsA_ref,
             epsT_ref, g1_ref, g2_ref, w_ref, out_ref):
    Rc = Rc_ref[0]
    Rsc = Rsc_ref[0]
    ac = ac_ref[0]
    asc = asc_ref[0]
    T0 = T_ref[0]
    T1 = T_ref[1]

    epsR = epsR_ref[...]
    epsA = epsA_ref[...]
    epsT = epsT_ref[...]

    R_s = jnp.exp(Rc) * jnp.exp(Rsc) * jnp.exp(0.1 * epsR)
    alpha_s = jnp.exp(ac) * jnp.exp(asc) * jnp.exp(0.1 * epsA)
    T_x0 = jnp.exp(T0)
    T_x1 = jnp.exp(T1)
    z = epsT + jnp.log(T_x0 / (T_x1 + EPS))
    T_samp = 1.0 / (1.0 + jnp.exp(-z))
    alpha_R = _log1mexp(alpha_s * R_s)
    lt = jnp.log1p(T_samp)

    eps1 = g1_ref[:, 0:S]
    a1 = g1_ref[:, S:S + 1]
    b1 = g1_ref[:, S + 1:S + 2]
    eps2 = g2_ref[:, 0:S]
    a2 = g2_ref[:, S:S + 1]
    b2 = g2_ref[:, S + 1:S + 2]

    s1 = jnp.exp(b1)
    s2 = jnp.exp(b2)
    hi = R_s - EPS
    r_i = jnp.minimum(jnp.maximum(a1 + s1 * eps1, EPS), hi)
    r_j = jnp.minimum(jnp.maximum(a2 + s2 * eps2, EPS), hi)

    alpha_r_i = _log1mexp(2.0 * alpha_s * r_i)
    a_R_ri = -alpha_s * (R_s - r_i)
    a_R_rj = -alpha_s * (R_s - r_j)

    s1e = s1 + EPS
    inv_s1 = 1.0 / s1e
    log_s1 = jnp.log(s1e)
    q = (-0.5 * ((r_i - a1) * inv_s1) ** 2 - log_s1
         - 0.5 * jnp.log(2.0 * jnp.pi))

    wf = w_ref[...].astype(jnp.float32)
    edges = jnp.where(wf > 0, 1.0, 0.0)
    out_ref[...] = (edges * (alpha_r_i + a_R_ri + a_R_rj)
                    - alpha_R + q - lt)


def _tc_compute(Rc, Rsc, ac, asc, T, epsR, epsA, epsT, g1, g2, w2d):
    grid = (L // LB,)
    return pl.pallas_call(
        _tc_body,
        grid=grid,
        in_specs=[
            pl.BlockSpec(memory_space=pltpu.SMEM),
            pl.BlockSpec(memory_space=pltpu.SMEM),
            pl.BlockSpec(memory_space=pltpu.SMEM),
            pl.BlockSpec(memory_space=pltpu.SMEM),
            pl.BlockSpec(memory_space=pltpu.SMEM),
            pl.BlockSpec((1, S), lambda i: (0, 0)),
            pl.BlockSpec((1, S), lambda i: (0, 0)),
            pl.BlockSpec((1, S), lambda i: (0, 0)),
            pl.BlockSpec((LB, D), lambda i: (i, 0)),
            pl.BlockSpec((LB, D), lambda i: (i, 0)),
            pl.BlockSpec((LB, 1), lambda i: (i, 0)),
        ],
        out_specs=pl.BlockSpec((LB, S), lambda i: (i, 0)),
        out_shape=jax.ShapeDtypeStruct((L, S), jnp.float32),
    )(Rc, Rsc, ac, asc, T, epsR, epsA, epsT, g1, g2, w2d)


def kernel(idx1, idx2, weights, rs_loc, rs_scale, phis_loc, phis_scale,
           R_conc, R_scale, T, alpha_conc, alpha_scale,
           eps_R, eps_T, eps_alpha, eps_r):
    idx1 = idx1.astype(jnp.int32)
    idx2 = idx2.astype(jnp.int32)
    table = _build_table(eps_r, rs_loc, rs_scale)
    g1, g2 = _sc_gather(table, idx1, idx2)
    w2d = weights.astype(jnp.bfloat16)[:, None]
    out_t = _tc_compute(R_conc.reshape(1), R_scale.reshape(1),
                        alpha_conc.reshape(1), alpha_scale.reshape(1), T,
                        eps_R[None, :], eps_alpha[None, :],
                        eps_T[None, :], g1, g2, w2d)
    return out_t.T


# speedup vs baseline: 2.6093x; 1.0049x over previous
import functools

import jax
import jax.numpy as jnp
from jax import lax
from jax.experimental import pallas as pl
from jax.experimental.pallas import tpu as pltpu
from jax.experimental.pallas import tpu_sc as plsc

N = 100000
S = 64
L = 16384
EPS = 1e-12
LN2 = 0.6931471805599453
D = 128
NW = 32
CHUNK = L // NW
SUB = 128
LB = 2048
TB = 256


def _log1mexp(x):
    x = jnp.maximum(x, 1e-10)
    em = jnp.where(x < 0.01,
                   x * (1.0 - x * (0.5 - x * (1.0 / 6.0))),
                   1.0 - jnp.exp(-x))
    return jnp.log(em + EPS)


NB = 8192
DW = 72


def _tbuild_body(eps_ref, rsl_ref, rss_ref, eye_ref, out_ref):
    x = jnp.concatenate([eps_ref[...],
                         rsl_ref[...].reshape(1, NB),
                         rss_ref[...].reshape(1, NB)], axis=0)
    out_ref[:, 0:DW] = lax.dot_general(
        x, eye_ref[...],
        dimension_numbers=(((0,), (0,)), ((), ())),
        preferred_element_type=jnp.float32,
        precision=lax.Precision.HIGHEST)


def _build_table(eps_r, rs_loc, rs_scale):
    eye = jnp.eye(S + 2, DW, dtype=jnp.float32)
    return pl.pallas_call(
        _tbuild_body,
        grid=(pl.cdiv(N, NB),),
        in_specs=[
            pl.BlockSpec((S, NB), lambda i: (0, i)),
            pl.BlockSpec((NB,), lambda i: (i,)),
            pl.BlockSpec((NB,), lambda i: (i,)),
            pl.BlockSpec((S + 2, DW), lambda i: (0, 0)),
        ],
        out_specs=pl.BlockSpec((NB, D), lambda i: (i, 0)),
        out_shape=jax.ShapeDtypeStruct((N, D), jnp.float32),
    )(eps_r, rs_loc, rs_scale, eye)


def _sc_gather_body(table_hbm, i1_hbm, i2_hbm, o1_hbm, o2_hbm,
                    idx1_v, idx2_v, rows_v, sem):
    wid = lax.axis_index("s") * 2 + lax.axis_index("c")
    base = wid * CHUNK
    pltpu.sync_copy(i1_hbm.at[pl.ds(base, CHUNK)], idx1_v)
    pltpu.sync_copy(i2_hbm.at[pl.ds(base, CHUNK)], idx2_v)
    copies = []
    for j in range(CHUNK // SUB):
        sl = pl.ds(j * SUB, SUB)
        copies.append(pltpu.async_copy(table_hbm.at[idx1_v.at[sl]],
                                       rows_v.at[sl], sem))
    for cp in copies:
        cp.wait()
    pltpu.sync_copy(rows_v, o1_hbm.at[pl.ds(base, CHUNK)])
    copies = []
    for j in range(CHUNK // SUB):
        sl = pl.ds(j * SUB, SUB)
        copies.append(pltpu.async_copy(table_hbm.at[idx2_v.at[sl]],
                                       rows_v.at[sl], sem))
    for cp in copies:
        cp.wait()
    pltpu.sync_copy(rows_v, o2_hbm.at[pl.ds(base, CHUNK)])


def _sc_gather(table, idx1, idx2):
    mesh = plsc.VectorSubcoreMesh(core_axis_name="c", subcore_axis_name="s")
    f = functools.partial(
        pl.kernel, mesh=mesh,
        out_type=(jax.ShapeDtypeStruct((L, D), jnp.float32),
                  jax.ShapeDtypeStruct((L, D), jnp.float32)),
        scratch_types=[
            pltpu.VMEM((CHUNK,), jnp.int32),
            pltpu.VMEM((CHUNK,), jnp.int32),
            pltpu.VMEM((CHUNK, D), jnp.float32),
            pltpu.SemaphoreType.DMA,
        ],
    )(_sc_gather_body)
    return f(table, idx1, idx2)


def _tc_body(Rc_ref, Rsc_ref, ac_ref, asc_ref, T_ref, epsR_ref, epsA_ref,
             epsT_ref, g1_ref, g2_ref, w_ref, eyeT_ref, out_ref):
    Rc = Rc_ref[0]
    Rsc = Rsc_ref[0]
    ac = ac_ref[0]
    asc = asc_ref[0]
    T0 = T_ref[0]
    T1 = T_ref[1]

    epsR = epsR_ref[...]
    epsA = epsA_ref[...]
    epsT = epsT_ref[...]

    R_s = jnp.exp(Rc) * jnp.exp(Rsc) * jnp.exp(0.1 * epsR)
    alpha_s = jnp.exp(ac) * jnp.exp(asc) * jnp.exp(0.1 * epsA)
    T_x0 = jnp.exp(T0)
    T_x1 = jnp.exp(T1)
    z = epsT + jnp.log(T_x0 / (T_x1 + EPS))
    T_samp = 1.0 / (1.0 + jnp.exp(-z))
    alpha_R = _log1mexp(alpha_s * R_s)
    lt = jnp.log1p(T_samp)

    eps1 = g1_ref[:, 0:S]
    a1 = g1_ref[:, S:S + 1]
    b1 = g1_ref[:, S + 1:S + 2]
    eps2 = g2_ref[:, 0:S]
    a2 = g2_ref[:, S:S + 1]
    b2 = g2_ref[:, S + 1:S + 2]

    s1 = jnp.exp(b1)
    s2 = jnp.exp(b2)
    hi = R_s - EPS
    r_i = jnp.minimum(jnp.maximum(a1 + s1 * eps1, EPS), hi)
    r_j = jnp.minimum(jnp.maximum(a2 + s2 * eps2, EPS), hi)

    alpha_r_i = _log1mexp(2.0 * alpha_s * r_i)
    a_R_ri = -alpha_s * (R_s - r_i)
    a_R_rj = -alpha_s * (R_s - r_j)

    s1e = s1 + EPS
    inv_s1 = 1.0 / s1e
    log_s1 = jnp.log(s1e)
    q = (-0.5 * ((r_i - a1) * inv_s1) ** 2 - log_s1
         - 0.5 * jnp.log(2.0 * jnp.pi))

    wf = w_ref[...].astype(jnp.float32)
    edges = jnp.where(wf > 0, 1.0, 0.0)
    res = (edges * (alpha_r_i + a_R_ri + a_R_rj)
           - alpha_R + q - lt)
    for k in range(LB // TB):
        out_ref[:, k * TB:(k + 1) * TB] = lax.dot_general(
            res[k * TB:(k + 1) * TB, :], eyeT_ref[...],
            dimension_numbers=(((0,), (0,)), ((), ())),
            preferred_element_type=jnp.float32,
            precision=lax.Precision.HIGHEST)


def _tc_compute(Rc, Rsc, ac, asc, T, epsR, epsA, epsT, g1, g2, w2d):
    grid = (L // LB,)
    return pl.pallas_call(
        _tc_body,
        grid=grid,
        in_specs=[
            pl.BlockSpec(memory_space=pltpu.SMEM),
            pl.BlockSpec(memory_space=pltpu.SMEM),
            pl.BlockSpec(memory_space=pltpu.SMEM),
            pl.BlockSpec(memory_space=pltpu.SMEM),
            pl.BlockSpec(memory_space=pltpu.SMEM),
            pl.BlockSpec((1, S), lambda i: (0, 0)),
            pl.BlockSpec((1, S), lambda i: (0, 0)),
            pl.BlockSpec((1, S), lambda i: (0, 0)),
            pl.BlockSpec((LB, D), lambda i: (i, 0)),
            pl.BlockSpec((LB, D), lambda i: (i, 0)),
            pl.BlockSpec((LB, 1), lambda i: (i, 0)),
            pl.BlockSpec((TB, TB), lambda i: (0, 0)),
        ],
        out_specs=pl.BlockSpec((S, LB), lambda i: (0, i)),
        out_shape=jax.ShapeDtypeStruct((S, L), jnp.float32),
    )(Rc, Rsc, ac, asc, T, epsR, epsA, epsT, g1, g2, w2d,
      jnp.eye(TB, TB, dtype=jnp.float32))


def kernel(idx1, idx2, weights, rs_loc, rs_scale, phis_loc, phis_scale,
           R_conc, R_scale, T, alpha_conc, alpha_scale,
           eps_R, eps_T, eps_alpha, eps_r):
    idx1 = idx1.astype(jnp.int32)
    idx2 = idx2.astype(jnp.int32)
    table = _build_table(eps_r, rs_loc, rs_scale)
    g1, g2 = _sc_gather(table, idx1, idx2)
    w2d = weights.astype(jnp.bfloat16)[:, None]
    return _tc_compute(R_conc.reshape(1), R_scale.reshape(1),
                       alpha_conc.reshape(1), alpha_scale.reshape(1), T,
                       eps_R[None, :], eps_alpha[None, :],
                       eps_T[None, :], g1, g2, w2d)


# speedup vs baseline: 2.6514x; 1.0162x over previous
import functools

import jax
import jax.numpy as jnp
from jax import lax
from jax.experimental import pallas as pl
from jax.experimental.pallas import tpu as pltpu
from jax.experimental.pallas import tpu_sc as plsc

N = 100000
S = 64
L = 16384
EPS = 1e-12
LN2 = 0.6931471805599453
D = 128
NW = 32
CHUNK = L // NW
SUB = 128
LB = 2048
TB = 128


def _log1mexp(x):
    x = jnp.maximum(x, 1e-10)
    em = jnp.where(x < 0.01,
                   x * (1.0 - x * (0.5 - x * (1.0 / 6.0))),
                   1.0 - jnp.exp(-x))
    return jnp.log(em + EPS)


NB = 8192
DW = 72


def _tbuild_body(eps_ref, rsl_ref, rss_ref, eye_ref, out_ref):
    x = jnp.concatenate([eps_ref[...],
                         rsl_ref[...].reshape(1, NB),
                         rss_ref[...].reshape(1, NB)], axis=0)
    out_ref[:, 0:DW] = lax.dot_general(
        x, eye_ref[...],
        dimension_numbers=(((0,), (0,)), ((), ())),
        preferred_element_type=jnp.float32,
        precision=lax.Precision.HIGHEST)


def _build_table(eps_r, rs_loc, rs_scale):
    eye = jnp.eye(S + 2, DW, dtype=jnp.float32)
    return pl.pallas_call(
        _tbuild_body,
        grid=(pl.cdiv(N, NB),),
        in_specs=[
            pl.BlockSpec((S, NB), lambda i: (0, i)),
            pl.BlockSpec((NB,), lambda i: (i,)),
            pl.BlockSpec((NB,), lambda i: (i,)),
            pl.BlockSpec((S + 2, DW), lambda i: (0, 0)),
        ],
        out_specs=pl.BlockSpec((NB, D), lambda i: (i, 0)),
        out_shape=jax.ShapeDtypeStruct((N, D), jnp.float32),
    )(eps_r, rs_loc, rs_scale, eye)


def _sc_gather_body(table_hbm, i1_hbm, i2_hbm, o1_hbm, o2_hbm,
                    idx1_v, idx2_v, rows_v, sem):
    wid = lax.axis_index("s") * 2 + lax.axis_index("c")
    base = wid * CHUNK
    pltpu.sync_copy(i1_hbm.at[pl.ds(base, CHUNK)], idx1_v)
    pltpu.sync_copy(i2_hbm.at[pl.ds(base, CHUNK)], idx2_v)
    copies = []
    for j in range(CHUNK // SUB):
        sl = pl.ds(j * SUB, SUB)
        copies.append(pltpu.async_copy(table_hbm.at[idx1_v.at[sl]],
                                       rows_v.at[sl], sem))
    for cp in copies:
        cp.wait()
    pltpu.sync_copy(rows_v, o1_hbm.at[pl.ds(base, CHUNK)])
    copies = []
    for j in range(CHUNK // SUB):
        sl = pl.ds(j * SUB, SUB)
        copies.append(pltpu.async_copy(table_hbm.at[idx2_v.at[sl]],
                                       rows_v.at[sl], sem))
    for cp in copies:
        cp.wait()
    pltpu.sync_copy(rows_v, o2_hbm.at[pl.ds(base, CHUNK)])


def _sc_gather(table, idx1, idx2):
    mesh = plsc.VectorSubcoreMesh(core_axis_name="c", subcore_axis_name="s")
    f = functools.partial(
        pl.kernel, mesh=mesh,
        out_type=(jax.ShapeDtypeStruct((L, D), jnp.float32),
                  jax.ShapeDtypeStruct((L, D), jnp.float32)),
        scratch_types=[
            pltpu.VMEM((CHUNK,), jnp.int32),
            pltpu.VMEM((CHUNK,), jnp.int32),
            pltpu.VMEM((CHUNK, D), jnp.float32),
            pltpu.SemaphoreType.DMA,
        ],
    )(_sc_gather_body)
    return f(table, idx1, idx2)


def _tc_body(Rc_ref, Rsc_ref, ac_ref, asc_ref, T_ref, epsR_ref, epsA_ref,
             epsT_ref, g1_ref, g2_ref, w_ref, eyeT_ref, out_ref):
    Rc = Rc_ref[0]
    Rsc = Rsc_ref[0]
    ac = ac_ref[0]
    asc = asc_ref[0]
    T0 = T_ref[0]
    T1 = T_ref[1]

    epsR = epsR_ref[...]
    epsA = epsA_ref[...]
    epsT = epsT_ref[...]

    R_s = jnp.exp(Rc) * jnp.exp(Rsc) * jnp.exp(0.1 * epsR)
    alpha_s = jnp.exp(ac) * jnp.exp(asc) * jnp.exp(0.1 * epsA)
    T_x0 = jnp.exp(T0)
    T_x1 = jnp.exp(T1)
    z = epsT + jnp.log(T_x0 / (T_x1 + EPS))
    T_samp = 1.0 / (1.0 + jnp.exp(-z))
    alpha_R = _log1mexp(alpha_s * R_s)
    lt = jnp.log1p(T_samp)

    eps1 = g1_ref[:, 0:S]
    a1 = g1_ref[:, S:S + 1]
    b1 = g1_ref[:, S + 1:S + 2]
    eps2 = g2_ref[:, 0:S]
    a2 = g2_ref[:, S:S + 1]
    b2 = g2_ref[:, S + 1:S + 2]

    s1 = jnp.exp(b1)
    s2 = jnp.exp(b2)
    hi = R_s - EPS
    r_i = jnp.minimum(jnp.maximum(a1 + s1 * eps1, EPS), hi)
    r_j = jnp.minimum(jnp.maximum(a2 + s2 * eps2, EPS), hi)

    alpha_r_i = _log1mexp(2.0 * alpha_s * r_i)
    a_R_ri = -alpha_s * (R_s - r_i)
    a_R_rj = -alpha_s * (R_s - r_j)

    s1e = s1 + EPS
    inv_s1 = 1.0 / s1e
    log_s1 = jnp.log(s1e)
    q = (-0.5 * ((r_i - a1) * inv_s1) ** 2 - log_s1
         - 0.5 * jnp.log(2.0 * jnp.pi))

    wf = w_ref[...].astype(jnp.float32)
    edges = jnp.where(wf > 0, 1.0, 0.0)
    res = (edges * (alpha_r_i + a_R_ri + a_R_rj)
           - alpha_R + q - lt)
    for k in range(LB // TB):
        out_ref[:, k * TB:(k + 1) * TB] = lax.dot_general(
            res[k * TB:(k + 1) * TB, :], eyeT_ref[...],
            dimension_numbers=(((0,), (0,)), ((), ())),
            preferred_element_type=jnp.float32,
            precision=lax.Precision.HIGHEST)


def _tc_compute(Rc, Rsc, ac, asc, T, epsR, epsA, epsT, g1, g2, w2d):
    grid = (L // LB,)
    return pl.pallas_call(
        _tc_body,
        grid=grid,
        in_specs=[
            pl.BlockSpec(memory_space=pltpu.SMEM),
            pl.BlockSpec(memory_space=pltpu.SMEM),
            pl.BlockSpec(memory_space=pltpu.SMEM),
            pl.BlockSpec(memory_space=pltpu.SMEM),
            pl.BlockSpec(memory_space=pltpu.SMEM),
            pl.BlockSpec((1, S), lambda i: (0, 0)),
            pl.BlockSpec((1, S), lambda i: (0, 0)),
            pl.BlockSpec((1, S), lambda i: (0, 0)),
            pl.BlockSpec((LB, D), lambda i: (i, 0)),
            pl.BlockSpec((LB, D), lambda i: (i, 0)),
            pl.BlockSpec((LB, 1), lambda i: (i, 0)),
            pl.BlockSpec((TB, TB), lambda i: (0, 0)),
        ],
        out_specs=pl.BlockSpec((S, LB), lambda i: (0, i)),
        out_shape=jax.ShapeDtypeStruct((S, L), jnp.float32),
    )(Rc, Rsc, ac, asc, T, epsR, epsA, epsT, g1, g2, w2d,
      jnp.eye(TB, TB, dtype=jnp.float32))


def kernel(idx1, idx2, weights, rs_loc, rs_scale, phis_loc, phis_scale,
           R_conc, R_scale, T, alpha_conc, alpha_scale,
           eps_R, eps_T, eps_alpha, eps_r):
    idx1 = idx1.astype(jnp.int32)
    idx2 = idx2.astype(jnp.int32)
    table = _build_table(eps_r, rs_loc, rs_scale)
    g1, g2 = _sc_gather(table, idx1, idx2)
    w2d = weights.astype(jnp.bfloat16)[:, None]
    return _tc_compute(R_conc.reshape(1), R_scale.reshape(1),
                       alpha_conc.reshape(1), alpha_scale.reshape(1), T,
                       eps_R[None, :], eps_alpha[None, :],
                       eps_T[None, :], g1, g2, w2d)
